# baseline (speedup 1.0000x reference)
import functools

import jax
import jax.numpy as jnp
from jax import lax
from jax.experimental import pallas as pl
from jax.experimental.pallas import tpu as pltpu
from jax.experimental.pallas import tpu_sc as plsc

B = 16384
D = 64
NEG = 20
NT = NEG + 1
SV = 1000
NCORES = 2
NSUB = 16
NW = NCORES * NSUB
BW = B // NW
C = 16
NCH = BW // C
GW = 80
NGB = C * NEG // GW
NP = NT * C + 1
PP = C + 1


def _sc_scores(cw2d, nb2d, neg2d, ctab, s1, s2, s3, s4, ntab, w_splat):
  mesh = plsc.VectorSubcoreMesh(core_axis_name="c", subcore_axis_name="s",
                                num_cores=NCORES, num_subcores=NSUB)

  @functools.partial(
      pl.kernel,
      mesh=mesh,
      out_type=jax.ShapeDtypeStruct((NW * 16,), jnp.float32),
      compiler_params=pltpu.CompilerParams(needs_layout_passes=False,
                                           use_tc_tiling_on_sc=False),
      scratch_types=[
          [pltpu.VMEM((C, 5), jnp.int32) for _ in range(2)],
          [pltpu.VMEM((C, 1), jnp.int32) for _ in range(2)],
          [pltpu.VMEM((C, NEG), jnp.int32) for _ in range(2)],
          [pltpu.VMEM((5 * C,), jnp.int32) for _ in range(2)],
          [pltpu.VMEM((C,), jnp.int32) for _ in range(2)],
          [pltpu.VMEM((C * NEG,), jnp.int32) for _ in range(2)],
          [[pltpu.VMEM((C, D), jnp.float32) for _ in range(5)]
           for _ in range(2)],
          [pltpu.VMEM((C, D), jnp.float32) for _ in range(2)],
          [pltpu.VMEM((C * NEG, D), jnp.float32) for _ in range(2)],
          pltpu.VMEM((D * NP,), jnp.float32),
          pltpu.VMEM((D * PP,), jnp.float32),
          pltpu.VMEM((16,), jnp.float32),
          pltpu.VMEM((5, 16), jnp.float32),
          [pltpu.SemaphoreType.DMA for _ in range(2)],
          [pltpu.SemaphoreType.DMA for _ in range(2)],
      ],
  )
  def k(cw_h, nb_h, neg_h, ct_h, s1_h, s2_h, s3_h, s4_h, nt_h, w_h, out_h,
        cwraw_v, nbraw_v, negraw_v, cwi_v, nbi_v, negi_v,
        srows_v, nbrows_v, negrows_v, rowst_v, poolt_v, acc_v, w_v,
        sem_i, sem_g):
    wid = lax.axis_index("s") * NCORES + lax.axis_index("c")
    pltpu.sync_copy(w_h, w_v)
    iota = lax.iota(jnp.int32, 16)
    tabs = (ct_h, s1_h, s2_h, s3_h, s4_h)
    qb_t = [iota * NP + q * 16 * NP for q in range(4)]
    qb_p = [iota * PP + q * 16 * PP for q in range(4)]

    def issue_idx(j, b):
      base = wid * BW + j * C
      pltpu.async_copy(cw_h.at[pl.ds(base, C)], cwraw_v[b], sem_i[b])
      pltpu.async_copy(nb_h.at[pl.ds(base, C)], nbraw_v[b], sem_i[b])
      pltpu.async_copy(neg_h.at[pl.ds(base, C)], negraw_v[b], sem_i[b])

    def wait_idx(b):
      pltpu.make_async_copy(cw_h.at[pl.ds(0, C)], cwraw_v[b],
                            sem_i[b]).wait()
      pltpu.make_async_copy(nb_h.at[pl.ds(0, C)], nbraw_v[b],
                            sem_i[b]).wait()
      pltpu.make_async_copy(neg_h.at[pl.ds(0, C)], negraw_v[b],
                            sem_i[b]).wait()

    def unpack_idx(b):
      for t in range(5):
        v = plsc.load_gather(cwraw_v[b], [iota, lax.broadcast(t, (16,))])
        cwi_v[b][pl.ds(t * C, 16)] = v
      v = plsc.load_gather(nbraw_v[b], [iota, lax.broadcast(0, (16,))])
      nbi_v[b][...] = v
      for n in range(NEG):
        v = plsc.load_gather(negraw_v[b], [iota, lax.broadcast(n, (16,))])
        negi_v[b][pl.ds(n * C, 16)] = v

    def issue_gathers(b):
      for t in range(5):
        pltpu.async_copy(tabs[t].at[cwi_v[b].at[pl.ds(t * C, C)]],
                         srows_v[b][t], sem_g[b])
      pltpu.async_copy(nt_h.at[nbi_v[b]], nbrows_v[b], sem_g[b])
      for r in range(NGB):
        pltpu.async_copy(nt_h.at[negi_v[b].at[pl.ds(r * GW, GW)]],
                         negrows_v[b].at[pl.ds(r * GW, GW)], sem_g[b])

    def wait_gathers(b):
      for t in range(5):
        pltpu.make_async_copy(tabs[t].at[pl.ds(0, C)], srows_v[b][t],
                              sem_g[b]).wait()
      pltpu.make_async_copy(nt_h.at[pl.ds(0, C)], nbrows_v[b],
                            sem_g[b]).wait()
      for r in range(NGB):
        pltpu.make_async_copy(nt_h.at[pl.ds(0, GW)],
                              negrows_v[b].at[pl.ds(r * GW, GW)],
                              sem_g[b]).wait()

    def softplus(y):
      t = jnp.exp(-jnp.abs(y))
      z = t / (2.0 + t)
      z2 = z * z
      p = z * (2.0 + z2 * (2.0 / 3.0 + z2 * (2.0 / 5.0 + z2 * (
          2.0 / 7.0 + z2 * (2.0 / 9.0)))))
      return jnp.maximum(y, 0.0) + p

    def compute(b):
      @plsc.parallel_loop(0, C, unroll=4)
      def _(p):
        for q in range(4):
          r = nbrows_v[b][p, pl.ds(q * 16, 16)]
          plsc.store_scatter(rowst_v, [qb_t[q] + p], r)

      @plsc.parallel_loop(0, C * NEG, unroll=8)
      def _(p):
        for q in range(4):
          r = negrows_v[b][p, pl.ds(q * 16, 16)]
          plsc.store_scatter(rowst_v, [qb_t[q] + (p + C)], r)

      wv = [w_v[t, :] for t in range(5)]

      @plsc.parallel_loop(0, C, unroll=4)
      def _(e):
        for q in range(4):
          acc = wv[0] * srows_v[b][0][e, pl.ds(q * 16, 16)]
          for t in range(1, 5):
            acc = acc + wv[t] * srows_v[b][t][e, pl.ds(q * 16, 16)]
          plsc.store_scatter(poolt_v, [qb_p[q] + e], acc)

      zero = jnp.zeros((16,), jnp.float32)

      @plsc.parallel_loop(0, D, unroll=4, carry=(zero,) * NT)
      def accs(d, acc_c):
        pv = poolt_v[pl.ds(d * PP, 16)]
        base = d * NP
        return tuple(acc_c[m] + pv * rowst_v[pl.ds(base + m * 16, 16)]
                     for m in range(NT))
      total = softplus(jnp.clip(-accs[0], -10.0, 10.0))
      for m in range(1, NT):
        total = total + softplus(jnp.clip(accs[m], -10.0, 10.0))
      acc_v[...] = acc_v[...] + total

    acc_v[...] = jnp.zeros((16,), jnp.float32)
    issue_idx(0, 0)
    issue_idx(1, 1)
    wait_idx(0)
    unpack_idx(0)
    issue_gathers(0)

    def outer(j0, carry):
      for bb in range(2):
        j = j0 * 2 + bb
        wait_gathers(bb)

        @pl.when(j + 1 < NCH)
        def _():
          wait_idx(1 - bb)
          unpack_idx(1 - bb)
          issue_gathers(1 - bb)

        @pl.when(j + 2 < NCH)
        def _():
          issue_idx(j + 2, bb)

        compute(bb)
      return carry

    lax.fori_loop(0, NCH // 2, outer, 0)
    pltpu.sync_copy(acc_v, out_h.at[pl.ds(wid * 16, 16)])

  return k(cw2d, nb2d, neg2d, ctab, s1, s2, s3, s4, ntab, w_splat)


def kernel(center_word, neighor_word, neg_word, center_table, neighbor_table,
           side1_table, side2_table, side3_table, side4_table,
           embedding_weight):
  w_splat = jnp.broadcast_to(
      embedding_weight.reshape(5, 1).astype(jnp.float32), (5, 16))
  partials = _sc_scores(center_word.astype(jnp.int32),
                        neighor_word.astype(jnp.int32),
                        neg_word.astype(jnp.int32), center_table[:SV],
                        neighbor_table, side1_table, side2_table,
                        side3_table, side4_table, w_splat)
  return jnp.sum(partials) * (1.0 / B)


# speedup vs baseline: 1.0063x; 1.0063x over previous
import functools

import jax
import jax.numpy as jnp
from jax import lax
from jax.experimental import pallas as pl
from jax.experimental.pallas import tpu as pltpu
from jax.experimental.pallas import tpu_sc as plsc

B = 16384
D = 64
NEG = 20
NT = NEG + 1
SV = 1000
NCORES = 2
NSUB = 16
NW = NCORES * NSUB
BW = B // NW
C = 16
NCH = BW // C
GW = 80
NGB = C * NEG // GW
NP = NT * C + 1
PP = C + 1


def _sc_scores(cw2d, nb2d, neg2d, ctab, s1, s2, s3, s4, ntab, w_splat):
  mesh = plsc.VectorSubcoreMesh(core_axis_name="c", subcore_axis_name="s",
                                num_cores=NCORES, num_subcores=NSUB)

  @functools.partial(
      pl.kernel,
      mesh=mesh,
      out_type=jax.ShapeDtypeStruct((NW * 16,), jnp.float32),
      compiler_params=pltpu.CompilerParams(needs_layout_passes=False,
                                           use_tc_tiling_on_sc=False),
      scratch_types=[
          [pltpu.VMEM((C, 5), jnp.int32) for _ in range(2)],
          [pltpu.VMEM((C, 1), jnp.int32) for _ in range(2)],
          [pltpu.VMEM((C, NEG), jnp.int32) for _ in range(2)],
          [pltpu.VMEM((5 * C,), jnp.int32) for _ in range(2)],
          [pltpu.VMEM((C,), jnp.int32) for _ in range(2)],
          [pltpu.VMEM((C * NEG,), jnp.int32) for _ in range(2)],
          [[pltpu.VMEM((C, D), jnp.float32) for _ in range(5)]
           for _ in range(2)],
          [pltpu.VMEM((C, D), jnp.float32) for _ in range(2)],
          [pltpu.VMEM((C * NEG, D), jnp.float32) for _ in range(2)],
          pltpu.VMEM((D * NP,), jnp.float32),
          pltpu.VMEM((D * PP,), jnp.float32),
          pltpu.VMEM((16,), jnp.float32),
          pltpu.VMEM((5, 16), jnp.float32),
          [pltpu.SemaphoreType.DMA for _ in range(2)],
          [pltpu.SemaphoreType.DMA for _ in range(2)],
      ],
  )
  def k(cw_h, nb_h, neg_h, ct_h, s1_h, s2_h, s3_h, s4_h, nt_h, w_h, out_h,
        cwraw_v, nbraw_v, negraw_v, cwi_v, nbi_v, negi_v,
        srows_v, nbrows_v, negrows_v, rowst_v, poolt_v, acc_v, w_v,
        sem_i, sem_g):
    wid = lax.axis_index("s") * NCORES + lax.axis_index("c")
    pltpu.sync_copy(w_h, w_v)
    iota = lax.iota(jnp.int32, 16)
    tabs = (ct_h, s1_h, s2_h, s3_h, s4_h)
    qb_t = [iota * NP + q * 16 * NP for q in range(4)]
    qb_p = [iota * PP + q * 16 * PP for q in range(4)]

    def issue_idx(j, b):
      base = wid * BW + j * C
      pltpu.async_copy(cw_h.at[pl.ds(base, C)], cwraw_v[b], sem_i[b])
      pltpu.async_copy(nb_h.at[pl.ds(base, C)], nbraw_v[b], sem_i[b])
      pltpu.async_copy(neg_h.at[pl.ds(base, C)], negraw_v[b], sem_i[b])

    def wait_idx(b):
      pltpu.make_async_copy(cw_h.at[pl.ds(0, C)], cwraw_v[b],
                            sem_i[b]).wait()
      pltpu.make_async_copy(nb_h.at[pl.ds(0, C)], nbraw_v[b],
                            sem_i[b]).wait()
      pltpu.make_async_copy(neg_h.at[pl.ds(0, C)], negraw_v[b],
                            sem_i[b]).wait()

    def unpack_idx(b):
      for t in range(5):
        v = plsc.load_gather(cwraw_v[b], [iota, lax.broadcast(t, (16,))])
        cwi_v[b][pl.ds(t * C, 16)] = v
      v = plsc.load_gather(nbraw_v[b], [iota, lax.broadcast(0, (16,))])
      nbi_v[b][...] = v
      for n in range(NEG):
        v = plsc.load_gather(negraw_v[b], [iota, lax.broadcast(n, (16,))])
        negi_v[b][pl.ds(n * C, 16)] = v

    def issue_gathers(b):
      for t in range(5):
        pltpu.async_copy(tabs[t].at[cwi_v[b].at[pl.ds(t * C, C)]],
                         srows_v[b][t], sem_g[b])
      pltpu.async_copy(nt_h.at[nbi_v[b]], nbrows_v[b], sem_g[b])
      for r in range(NGB):
        pltpu.async_copy(nt_h.at[negi_v[b].at[pl.ds(r * GW, GW)]],
                         negrows_v[b].at[pl.ds(r * GW, GW)], sem_g[b])

    def wait_gathers(b):
      for t in range(5):
        pltpu.make_async_copy(tabs[t].at[pl.ds(0, C)], srows_v[b][t],
                              sem_g[b]).wait()
      pltpu.make_async_copy(nt_h.at[pl.ds(0, C)], nbrows_v[b],
                            sem_g[b]).wait()
      for r in range(NGB):
        pltpu.make_async_copy(nt_h.at[pl.ds(0, GW)],
                              negrows_v[b].at[pl.ds(r * GW, GW)],
                              sem_g[b]).wait()

    def softplus(y):
      t = jnp.exp(-jnp.abs(y))
      z = t / (2.0 + t)
      z2 = z * z
      p = z * (2.0 + z2 * (2.0 / 3.0 + z2 * (2.0 / 5.0 + z2 * (
          2.0 / 7.0 + z2 * (2.0 / 9.0)))))
      return jnp.maximum(y, 0.0) + p

    def compute(b):
      @plsc.parallel_loop(0, C, unroll=4)
      def _(p):
        for q in range(4):
          r = nbrows_v[b][p, pl.ds(q * 16, 16)]
          plsc.store_scatter(rowst_v, [qb_t[q] + p], r)

      @plsc.parallel_loop(0, C * NEG, unroll=8)
      def _(p):
        for q in range(4):
          r = negrows_v[b][p, pl.ds(q * 16, 16)]
          plsc.store_scatter(rowst_v, [qb_t[q] + (p + C)], r)

      wv = [w_v[t, :] for t in range(5)]

      @plsc.parallel_loop(0, C, unroll=4)
      def _(e):
        for q in range(4):
          acc = wv[0] * srows_v[b][0][e, pl.ds(q * 16, 16)]
          for t in range(1, 5):
            acc = acc + wv[t] * srows_v[b][t][e, pl.ds(q * 16, 16)]
          plsc.store_scatter(poolt_v, [qb_p[q] + e], acc)

      def dot_d(d, accs):
        pv = poolt_v[pl.ds(d * PP, 16)]
        base = d * NP
        return tuple(accs[m] + pv * rowst_v[pl.ds(base + m * 16, 16)]
                     for m in range(NT))

      zero = jnp.zeros((16,), jnp.float32)
      accs = lax.fori_loop(0, D, dot_d, (zero,) * NT)
      total = softplus(jnp.clip(-accs[0], -10.0, 10.0))
      for m in range(1, NT):
        total = total + softplus(jnp.clip(accs[m], -10.0, 10.0))
      acc_v[...] = acc_v[...] + total

    acc_v[...] = jnp.zeros((16,), jnp.float32)
    issue_idx(0, 0)
    issue_idx(1, 1)
    wait_idx(0)
    unpack_idx(0)
    issue_gathers(0)

    def outer(j0, carry):
      for bb in range(2):
        j = j0 * 2 + bb
        wait_gathers(bb)

        @pl.when(j + 1 < NCH)
        def _():
          wait_idx(1 - bb)
          unpack_idx(1 - bb)
          issue_gathers(1 - bb)

        @pl.when(j + 2 < NCH)
        def _():
          issue_idx(j + 2, bb)

        compute(bb)
      return carry

    lax.fori_loop(0, NCH // 2, outer, 0)
    pltpu.sync_copy(acc_v, out_h.at[pl.ds(wid * 16, 16)])

  return k(cw2d, nb2d, neg2d, ctab, s1, s2, s3, s4, ntab, w_splat)


def kernel(center_word, neighor_word, neg_word, center_table, neighbor_table,
           side1_table, side2_table, side3_table, side4_table,
           embedding_weight):
  w_splat = jnp.broadcast_to(
      embedding_weight.reshape(5, 1).astype(jnp.float32), (5, 16))
  partials = _sc_scores(center_word.astype(jnp.int32),
                        neighor_word.astype(jnp.int32),
                        neg_word.astype(jnp.int32), center_table[:SV],
                        neighbor_table, side1_table, side2_table,
                        side3_table, side4_table, w_splat)
  return jnp.sum(partials) * (1.0 / B)
